# baseline (speedup 1.0000x reference)
import functools

import numpy as np

import jax
import jax.numpy as jnp
from jax import lax
from jax.experimental import pallas as pl
from jax.experimental.pallas import tpu as pltpu
from jax.experimental.pallas import tpu_sc as plsc

VOCAB = 1000000
HIDDEN = 128
B, L = 1024, 200
CHUNK = 40
NCHUNKS = (B * L) // CHUNK
NW = 32
CPW = NCHUNKS // NW
TPW = CPW * CHUNK
LANES = 16
NJ = HIDDEN // LANES


def _rsqrt(v):
    vi = lax.bitcast_convert_type(v, jnp.int32)
    yi = jnp.int32(0x5F3759DF) - (vi >> 1)
    y = lax.bitcast_convert_type(yi, jnp.float32)
    for _ in range(2):
        y = y * (1.5 - 0.5 * v * y * y)
    return y


_DNUMS = lax.GatherDimensionNumbers(
    offset_dims=(), collapsed_slice_dims=(0,), start_index_map=(0,))


def _perm(v, idx):
    return lax.gather(v, idx[:, None], _DNUMS, (1,),
                      mode=lax.GatherScatterMode.PROMISE_IN_BOUNDS)


def _splat_lane(v, lanes, lane):
    return _perm(v, lanes * 0 + lane)


def _merge4(vs, lanes):
    m8 = (lanes & 8) == 0
    m4 = (lanes & 4) == 0
    ab = jnp.where(m8, vs[0] + _perm(vs[0], lanes ^ 8),
                   vs[1] + _perm(vs[1], lanes ^ 8))
    cd = jnp.where(m8, vs[2] + _perm(vs[2], lanes ^ 8),
                   vs[3] + _perm(vs[3], lanes ^ 8))
    q = jnp.where(m4, ab + _perm(ab, lanes ^ 4),
                  cd + _perm(cd, lanes ^ 4))
    q = q + _perm(q, lanes ^ 2)
    q = q + _perm(q, lanes ^ 1)
    return q


_BLANE = (0, 8, 4, 12)


def _body(ids_hbm, tt_hbm, word_hbm, pos_hbm, typ_hbm, gamma_hbm, beta_hbm,
          out_hbm, ids_v, tt_v, rows0, rows1, sbuf0, sbuf1, addpt_v, typ_v,
          gamma_v, beta_v, gsem0, gsem1, osem0, osem1):
    wid = lax.axis_index("s") * 2 + lax.axis_index("c")
    tok0 = wid * TPW

    pltpu.sync_copy(ids_hbm.at[pl.ds(tok0, TPW)], ids_v)
    pltpu.sync_copy(tt_hbm.at[pl.ds(tok0, TPW)], tt_v.at[pl.ds(0, TPW)])


    pltpu.sync_copy(pos_hbm.at[pl.ds(0, L)], addpt_v.at[pl.ds(0, L)])
    pltpu.sync_copy(pos_hbm.at[pl.ds(0, L)], addpt_v.at[pl.ds(L, L)])
    pltpu.sync_copy(typ_hbm, typ_v)
    pltpu.sync_copy(gamma_hbm, gamma_v)
    pltpu.sync_copy(beta_hbm, beta_v)

    def fold(r, _):
        for j in range(NJ):
            js = pl.ds(j * LANES, LANES)
            addpt_v[r, js] = addpt_v[r, js] + typ_v[0, js]
            addpt_v[L + r, js] = addpt_v[L + r, js] + typ_v[1, js]
        return 0

    lax.fori_loop(0, L, fold, 0)

    inv_h = jnp.float32(1.0 / HIDDEN)
    lanes = lax.iota(jnp.int32, LANES)

    def half_step(c, rows_cur, rows_nxt, sbuf_cur, gsem_cur, gsem_nxt,
                  osem_cur):
        @pl.when(c >= CPW)
        def _():
            nbase = pl.multiple_of((c + 1) * CHUNK, CHUNK)
            pltpu.async_copy(
                word_hbm.at[ids_v.at[pl.ds(nbase, CHUNK)]], rows_nxt,
                gsem_nxt)

        @pl.when(c >= 2)
        def _():
            pltpu.make_async_copy(
                sbuf_cur, out_hbm.at[pl.ds(tok0, CHUNK)], osem_cur).wait()

        pbase = lax.rem(c, jnp.int32(L // CHUNK)) * CHUNK
        cbase = pl.multiple_of(c * CHUNK, CHUNK)

        for tg in range(3):
            sg = tg * 16
            nt = 16 if tg < 2 else CHUNK - 32
            tt16 = tt_v[pl.ds(cbase + sg, LANES)]
            for gg in range(nt // 4):
                toks = [sg + gg * 4 + i for i in range(4)]
                xs_all, accs, acc2s = [], [], []
                for t in toks:
                    arow = tt16[t - sg] * L + (pbase + t)
                    xs = []
                    acc = None
                    acc2 = None
                    for j in range(NJ):
                        js = pl.ds(j * LANES, LANES)
                        x = rows_cur[t, js] + addpt_v[arow, js]
                        xs.append(x)
                        x2 = x * x
                        acc = x if acc is None else acc + x
                        acc2 = x2 if acc2 is None else acc2 + x2
                    xs_all.append(xs)
                    accs.append(acc)
                    acc2s.append(acc2)
                meanv = _merge4(accs, lanes) * inv_h
                varv = _merge4(acc2s, lanes) * inv_h - meanv * meanv
                rstdv = _rsqrt(varv + 1e-12)
                offv = meanv * rstdv
                for i, t in enumerate(toks):
                    rs = _splat_lane(rstdv, lanes, _BLANE[i])
                    of = _splat_lane(offv, lanes, _BLANE[i])
                    for j in range(NJ):
                        js = pl.ds(j * LANES, LANES)
                        sbuf_cur[t, js] = xs_all[i][j] * rs - of

        pltpu.async_copy(sbuf_cur, out_hbm.at[pl.ds(tok0 + cbase, CHUNK)],
                         osem_cur)

    def pair(i, _):
        c0 = i * 2
        half_step(c0, rows0, rows1, sbuf0, gsem0, gsem1, osem0)
        half_step(c0 + 1, rows1, rows0, sbuf1, gsem1, gsem0, osem1)
        return 0

    lax.fori_loop(0, CPW // 2, pair, 0)

    pltpu.make_async_copy(
        sbuf0, out_hbm.at[pl.ds(tok0, CHUNK)], osem0).wait()
    pltpu.make_async_copy(
        sbuf1, out_hbm.at[pl.ds(tok0, CHUNK)], osem1).wait()


@jax.jit
def _sc_embed(ids1, tt1, word_emb, pos_emb, type_emb, gamma, beta):
    mesh = plsc.VectorSubcoreMesh(core_axis_name="c", subcore_axis_name="s")
    f = functools.partial(
        pl.kernel,
        out_type=jax.ShapeDtypeStruct((B * L, HIDDEN), jnp.float32),
        mesh=mesh,
        scratch_types=[
            pltpu.VMEM((TPW,), jnp.int32),
            pltpu.VMEM((TPW + LANES,), jnp.int32),
            pltpu.VMEM((CHUNK, HIDDEN), jnp.float32),
            pltpu.VMEM((CHUNK, HIDDEN), jnp.float32),
            pltpu.VMEM((CHUNK, HIDDEN), jnp.float32),
            pltpu.VMEM((CHUNK, HIDDEN), jnp.float32),
            pltpu.VMEM((2 * L, HIDDEN), jnp.float32),
            pltpu.VMEM((2, HIDDEN), jnp.float32),
            pltpu.VMEM((HIDDEN,), jnp.float32),
            pltpu.VMEM((HIDDEN,), jnp.float32),
            pltpu.SemaphoreType.DMA,
            pltpu.SemaphoreType.DMA,
            pltpu.SemaphoreType.DMA,
            pltpu.SemaphoreType.DMA,
        ],
    )(_body)
    return f(ids1, tt1, word_emb, pos_emb, type_emb, gamma, beta)


def kernel(input_ids, token_type_ids, word_emb, pos_emb, type_emb, gamma,
           beta):
    ids1 = input_ids.reshape(B * L)
    tt1 = token_type_ids.reshape(B * L)
    out = _sc_embed(ids1, tt1, word_emb, pos_emb, type_emb, gamma, beta)
    return out.reshape(B, L, HIDDEN)


# speedup vs baseline: 1.4787x; 1.4787x over previous
import functools

import jax
import jax.numpy as jnp
from jax import lax
from jax.experimental import pallas as pl
from jax.experimental.pallas import tpu as pltpu
from jax.experimental.pallas import tpu_sc as plsc

VOCAB = 1000000
HIDDEN = 128
B, L = 1024, 200
CHUNK = 40
NW = 32
NSLICES = 4
BSL = B // NSLICES
TSL = BSL * L
CPWS = TSL // (CHUNK * NW)
TPWS = CPWS * CHUNK
NBUF = 4
BB = 32


def _sc_body(ids_hbm, word_hbm, raw_hbm, ids_v, rows, gsems, osems):
    wid = lax.axis_index("s") * 2 + lax.axis_index("c")
    tok0 = wid * TPWS

    pltpu.sync_copy(ids_hbm.at[pl.ds(tok0, TPWS)], ids_v)

    for b in range(NBUF - 1):
        pltpu.async_copy(
            word_hbm.at[ids_v.at[pl.ds(b * CHUNK, CHUNK)]], rows[b],
            gsems[b])

    def step(c, b):
        b2 = (b + NBUF - 1) % NBUF

        @pl.when(c + NBUF - 1 < CPWS)
        def _():
            @pl.when(c > 0)
            def _():
                pltpu.make_async_copy(
                    rows[b2], raw_hbm.at[pl.ds(tok0, CHUNK)],
                    osems[b2]).wait()

            nbase = pl.multiple_of((c + NBUF - 1) * CHUNK, CHUNK)
            pltpu.async_copy(
                word_hbm.at[ids_v.at[pl.ds(nbase, CHUNK)]], rows[b2],
                gsems[b2])

        pltpu.make_async_copy(
            word_hbm.at[pl.ds(0, CHUNK)], rows[b], gsems[b]).wait()
        cbase = pl.multiple_of(c * CHUNK, CHUNK)
        pltpu.async_copy(rows[b], raw_hbm.at[pl.ds(tok0 + cbase, CHUNK)],
                         osems[b])

    def ring(i, _):
        c0 = i * NBUF
        for b in range(NBUF):
            step(c0 + b, b)
        return 0

    lax.fori_loop(0, CPWS // NBUF, ring, 0)

    for b in range(NBUF):
        pltpu.make_async_copy(
            rows[b], raw_hbm.at[pl.ds(tok0, CHUNK)], osems[b]).wait()


def _sc_gather(ids_slice, word_emb):
    mesh = plsc.VectorSubcoreMesh(core_axis_name="c", subcore_axis_name="s")

    def body(ids_hbm, word_hbm, raw_hbm, *scr):
        _sc_body(ids_hbm, word_hbm, raw_hbm, scr[0],
                 list(scr[1:1 + NBUF]), list(scr[1 + NBUF:1 + 2 * NBUF]),
                 list(scr[1 + 2 * NBUF:]))

    f = functools.partial(
        pl.kernel,
        out_type=jax.ShapeDtypeStruct((TSL, HIDDEN), jnp.float32),
        mesh=mesh,
        scratch_types=(
            [pltpu.VMEM((TPWS,), jnp.int32)]
            + [pltpu.VMEM((CHUNK, HIDDEN), jnp.float32)] * NBUF
            + [pltpu.SemaphoreType.DMA] * (2 * NBUF)
        ),
    )(body)
    return f(ids_slice, word_emb)


def _tc_body(raw_ref, tt_ref, pos_ref, typ_ref, gamma_ref, beta_ref,
             out_ref):
    x = raw_ref[...]
    ttf = tt_ref[...].astype(jnp.float32)
    pos = pos_ref[...]
    typ0 = typ_ref[0]
    typd = typ_ref[1] - typ0
    x = x + pos[None] + typ0[None, None] + ttf[..., None] * typd[None, None]
    mean = jnp.mean(x, axis=-1, keepdims=True)
    var = jnp.mean(x * x, axis=-1, keepdims=True) - mean * mean
    y = (x - mean) * lax.rsqrt(var + 1e-12)
    out_ref[...] = y * gamma_ref[...][None, None] + beta_ref[...][None, None]


def _tc_ln(raw_slice, tt_slice, pos_emb, type_emb, gamma, beta):
    grid = (BSL // BB,)
    return pl.pallas_call(
        _tc_body,
        grid=grid,
        in_specs=[
            pl.BlockSpec((BB, L, HIDDEN), lambda i: (i, 0, 0)),
            pl.BlockSpec((BB, L), lambda i: (i, 0)),
            pl.BlockSpec((L, HIDDEN), lambda i: (0, 0)),
            pl.BlockSpec((2, HIDDEN), lambda i: (0, 0)),
            pl.BlockSpec((HIDDEN,), lambda i: (0,)),
            pl.BlockSpec((HIDDEN,), lambda i: (0,)),
        ],
        out_specs=pl.BlockSpec((BB, L, HIDDEN), lambda i: (i, 0, 0)),
        out_shape=jax.ShapeDtypeStruct((BSL, L, HIDDEN), jnp.float32),
    )(raw_slice, tt_slice, pos_emb[:L], type_emb, gamma, beta)


@jax.jit
def _embed(input_ids, token_type_ids, word_emb, pos_emb, type_emb, gamma,
           beta):
    ids1 = input_ids.reshape(B * L)
    raws = [
        _sc_gather(lax.dynamic_slice_in_dim(ids1, s * TSL, TSL), word_emb)
        for s in range(NSLICES)
    ]
    outs = [
        _tc_ln(raws[s].reshape(BSL, L, HIDDEN),
               lax.dynamic_slice_in_dim(token_type_ids, s * BSL, BSL),
               pos_emb, type_emb, gamma, beta)
        for s in range(NSLICES)
    ]
    return jnp.concatenate(outs, axis=0)


def kernel(input_ids, token_type_ids, word_emb, pos_emb, type_emb, gamma,
           beta):
    return _embed(input_ids, token_type_ids, word_emb, pos_emb, type_emb,
                  gamma, beta)
